# baseline (speedup 1.0000x reference)
import functools

import jax
import jax.numpy as jnp
from jax import lax
from jax.experimental import pallas as pl
from jax.experimental.pallas import tpu as pltpu
from jax.experimental.pallas import tpu_sc as plsc

B, S, N = 8, 512, 2048
LOW_C, HIGH_C = 256, 128
MID = HIGH_C // 4
POS_H = 64
ATTN_M = 4
K_NEI = 16
NBLK = 256
NROWB = N // NBLK

_INF = float('inf')


def _interp_body(plow_ref, upw_ref, upb_ref, xlr_ref, xh_ref, ph_ref, out_ref):
    plf = jnp.maximum(
        jnp.dot(upw_ref[...], plow_ref[...], preferred_element_type=jnp.float32)
        + upb_ref[...], 0.0)
    xlr = xlr_ref[...]
    xh = xh_ref[...]
    rn = jnp.sum(xlr * xlr, axis=1, keepdims=True)
    cn = jnp.sum(xh * xh, axis=0, keepdims=True)
    d2 = rn + cn - 2.0 * jnp.dot(xlr, xh, preferred_element_type=jnp.float32)
    row = lax.broadcasted_iota(jnp.int32, (S, N), 0)
    wt_u = jnp.zeros((S, N), jnp.float32)
    norm = jnp.zeros((1, N), jnp.float32)
    for _ in range(3):
        m = jnp.min(d2, axis=0, keepdims=True)
        idx = jnp.min(jnp.where(d2 == m, row, S), axis=0, keepdims=True)
        sel = row == idx
        d2 = jnp.where(sel, _INF, d2)
        rec = 1.0 / (m + 1e-8)
        norm = norm + rec
        wt_u = wt_u + jnp.where(sel, rec, 0.0)
    wt = wt_u / norm
    out_ref[...] = (
        jnp.dot(plf, wt, preferred_element_type=jnp.float32) + ph_ref[...])


def _interp(plow, upw, upb, xlr, xh, ph):
    return pl.pallas_call(
        _interp_body,
        grid=(B,),
        in_specs=[
            pl.BlockSpec((None, LOW_C, S), lambda b: (b, 0, 0)),
            pl.BlockSpec((HIGH_C, LOW_C), lambda b: (0, 0)),
            pl.BlockSpec((HIGH_C, 1), lambda b: (0, 0)),
            pl.BlockSpec((None, S, 3), lambda b: (b, 0, 0)),
            pl.BlockSpec((None, 3, N), lambda b: (b, 0, 0)),
            pl.BlockSpec((None, HIGH_C, N), lambda b: (b, 0, 0)),
        ],
        out_specs=pl.BlockSpec((None, HIGH_C, N), lambda b: (b, 0, 0)),
        out_shape=jax.ShapeDtypeStruct((B, HIGH_C, N), jnp.float32),
    )(plow, upw, upb, xlr, xh, ph)


KNB = 256


def _knn_body(xb_ref, xf_ref, out_ref):
    b = pl.program_id(0)
    xb = xb_ref[...]
    xf = xf_ref[...]
    rn = jnp.sum(xb * xb, axis=1, keepdims=True)
    cn = jnp.sum(xf * xf, axis=0, keepdims=True)
    d2 = rn + cn - 2.0 * jnp.dot(xb, xf, preferred_element_type=jnp.float32)
    lane = lax.broadcasted_iota(jnp.int32, (KNB, N), 1)
    bits = lax.bitcast_convert_type(d2, jnp.int32)
    packed = lax.bitcast_convert_type((bits & ~2047) | lane, jnp.float32)
    cols = []
    for _ in range(K_NEI):
        m = jnp.min(packed, axis=1, keepdims=True)
        packed = jnp.where(packed == m, _INF, packed)
        cols.append(lax.bitcast_convert_type(m, jnp.int32) & 2047)
    idx = jnp.concatenate(cols, axis=1)
    out_ref[...] = idx + b * N


def _knn(xhT, xf, h):
    hb = N // 2 // KNB
    return pl.pallas_call(
        _knn_body,
        grid=(B, hb),
        in_specs=[
            pl.BlockSpec((None, KNB, 3), lambda b, r: (b, h * hb + r, 0)),
            pl.BlockSpec((None, 3, N), lambda b, r: (b, 0, 0)),
        ],
        out_specs=pl.BlockSpec((None, KNB, K_NEI), lambda b, r: (b, r, 0)),
        out_shape=jax.ShapeDtypeStruct((B, N // 2, K_NEI), jnp.int32),
    )(xhT, xf)


def _qkv_body(pts_ref, xhp_ref, bw_ref, bb_ref, qkvw_ref, q_ref, kvx_ref):
    x = (jnp.dot(bw_ref[...], pts_ref[...], preferred_element_type=jnp.float32)
         + bb_ref[...])
    qkv = jnp.dot(qkvw_ref[...], x, preferred_element_type=jnp.float32)
    q_ref[...] = qkv[:MID, :]
    kvt = jnp.transpose(qkv[MID:, :])
    kvx_ref[...] = jnp.concatenate(
        [kvt, xhp_ref[...], jnp.zeros((N, 128 - 2 * MID - 16), jnp.float32)],
        axis=1)


def _qkv(pts, xhp, beforew, beforeb, qkvw):
    return pl.pallas_call(
        _qkv_body,
        grid=(B,),
        in_specs=[
            pl.BlockSpec((None, HIGH_C, N), lambda b: (b, 0, 0)),
            pl.BlockSpec((None, N, 16), lambda b: (b, 0, 0)),
            pl.BlockSpec((MID, HIGH_C), lambda b: (0, 0)),
            pl.BlockSpec((MID, 1), lambda b: (0, 0)),
            pl.BlockSpec((3 * MID, MID), lambda b: (0, 0)),
        ],
        out_specs=[
            pl.BlockSpec((None, MID, N), lambda b: (b, 0, 0)),
            pl.BlockSpec((None, N, 128), lambda b: (b, 0, 0)),
        ],
        out_shape=[
            jax.ShapeDtypeStruct((B, MID, N), jnp.float32),
            jax.ShapeDtypeStruct((B, N, 128), jnp.float32),
        ],
    )(pts, xhp, beforew, beforeb, qkvw)


def _sc_gather(table, idx, d):
    m_tot = idx.shape[0]
    info = plsc.get_sparse_core_info()
    nw = info.num_cores * info.num_subcores
    per_w = m_tot // nw
    sub = 128
    nf = 4
    ch = sub * nf
    n_ch = per_w // ch
    mesh = plsc.VectorSubcoreMesh(core_axis_name="c", subcore_axis_name="s")

    @functools.partial(
        pl.kernel, mesh=mesh,
        compiler_params=pltpu.CompilerParams(use_tc_tiling_on_sc=True),
        out_type=jax.ShapeDtypeStruct((m_tot, d), jnp.float32),
        scratch_types=[
            pltpu.VMEM((ch,), jnp.int32),
            pltpu.VMEM((ch, d), jnp.float32),
            pltpu.SemaphoreType.DMA,
        ],
    )
    def k(table_hbm, idx_hbm, out_hbm, idx_v, rows_v, sem):
        wid = lax.axis_index("s") * info.num_cores + lax.axis_index("c")
        base = wid * per_w

        def body(i, carry):
            off = base + i * ch
            pltpu.sync_copy(idx_hbm.at[pl.ds(off, ch)], idx_v)
            copies = [
                pltpu.async_copy(table_hbm.at[idx_v.at[pl.ds(f * sub, sub)]],
                                 rows_v.at[pl.ds(f * sub, sub)], sem)
                for f in range(nf)
            ]
            for c in copies:
                c.wait()
            pltpu.sync_copy(rows_v, out_hbm.at[pl.ds(off, ch)])
            return carry

        lax.fori_loop(0, n_ch, body, 0)

    return k(table, idx)


def _attn_body(pts_ref, q_ref, g_ref, xhp_ref,
               pw1_ref, pb1_ref, pw2_ref, pb2_ref,
               aw1_ref, ab1_ref, aw2_ref, ab2_ref,
               afw_ref, afb_ref, out_ref):
    rows = NBLK * K_NEI
    g = g_ref[...]
    kg = g[:, :MID]
    vg = g[:, MID:2 * MID]
    xyzg = g[:, 2 * MID:2 * MID + 16]
    xh_rep = jnp.broadcast_to(
        xhp_ref[...][:, None, :], (NBLK, K_NEI, 16)).reshape(rows, 16)
    qt = jnp.transpose(q_ref[...])
    q_rep = jnp.broadcast_to(
        qt[:, None, :], (NBLK, K_NEI, MID)).reshape(rows, MID)
    dxyz = xh_rep - xyzg
    h = jnp.maximum(
        jnp.dot(dxyz, pw1_ref[...], preferred_element_type=jnp.float32)
        + pb1_ref[...], 0.0)
    rel = (jnp.dot(h, pw2_ref[...], preferred_element_type=jnp.float32)
           + pb2_ref[...])
    sim_in = q_rep - kg + rel
    bf = jnp.bfloat16
    hid = jnp.maximum(
        jnp.dot(sim_in.astype(bf), aw1_ref[...].astype(bf),
                preferred_element_type=jnp.float32)
        + ab1_ref[...], 0.0)
    sim = (jnp.dot(hid.astype(bf), aw2_ref[...].astype(bf),
                   preferred_element_type=jnp.float32)
           + ab2_ref[...])
    e = jnp.exp(sim)
    vr = vg + rel
    num = jnp.sum((e * vr).reshape(NBLK, K_NEI, MID), axis=1)
    den = jnp.sum(e.reshape(NBLK, K_NEI, MID), axis=1)
    agg = jnp.transpose(num / den)
    out_ref[...] = (
        pts_ref[...]
        + jnp.dot(afw_ref[...], agg, preferred_element_type=jnp.float32)
        + afb_ref[...])


def _attn(pts, q, gat, xhp, w, start_blk, nblk):
    rows = NBLK * K_NEI

    def full(shp):
        return pl.BlockSpec(shp, lambda b, r: tuple(0 for _ in shp))

    return pl.pallas_call(
        _attn_body,
        grid=(B, nblk),
        in_specs=[
            pl.BlockSpec((None, HIGH_C, NBLK),
                         lambda b, r: (b, 0, start_blk + r)),
            pl.BlockSpec((None, MID, NBLK),
                         lambda b, r: (b, 0, start_blk + r)),
            pl.BlockSpec((rows, 128), lambda b, r: (b * nblk + r, 0)),
            pl.BlockSpec((None, NBLK, 16),
                         lambda b, r: (b, start_blk + r, 0)),
            full((16, POS_H)), full((1, POS_H)),
            full((POS_H, MID)), full((1, MID)),
            full((MID, MID * ATTN_M)), full((1, MID * ATTN_M)),
            full((MID * ATTN_M, MID)), full((1, MID)),
            full((HIGH_C, MID)), full((HIGH_C, 1)),
        ],
        out_specs=pl.BlockSpec((None, HIGH_C, NBLK), lambda b, r: (b, 0, r)),
        out_shape=jax.ShapeDtypeStruct((B, HIGH_C, nblk * NBLK), jnp.float32),
    )(pts, q, gat, xhp,
      w['pw1'], w['pb1'], w['pw2'], w['pb2'],
      w['aw1'], w['ab1'], w['aw2'], w['ab2'], w['afw'], w['afb'])


def kernel(xyz_low, xyz_high, points_low, points_high, params):
    f32 = jnp.float32
    xhT = jnp.transpose(xyz_high, (0, 2, 1))
    xlr = jnp.transpose(xyz_low, (0, 2, 1))

    inv = 1.0 / jnp.sqrt(f32(1.0 + 1e-5))
    scale = inv * params['up_bn_g']
    upw = params['up_w'] * scale[:, None]
    upb = (params['up_b'] * scale + params['up_bn_b']).reshape(HIGH_C, 1)

    points = _interp(points_low, upw, upb, xlr, xyz_high, points_high)

    xhp = jnp.pad(xhT, ((0, 0), (0, 0), (0, 13)))
    gidx_h = [_knn(xhT, xyz_high, h) for h in (0, 1)]

    def part_idx(p, nsplit):
        per = N // nsplit
        h, o = divmod(p * per, N // 2)
        return gidx_h[h][:, o:o + per, :].reshape(B * per * K_NEI)

    for blk_i, blk in enumerate(params['blocks']):
        w = {
            'pw1': jnp.pad(blk['pos_w1'], ((0, 0), (0, 13))).T,
            'pb1': blk['pos_b1'].reshape(1, POS_H),
            'pw2': blk['pos_w2'].T, 'pb2': blk['pos_b2'].reshape(1, MID),
            'aw1': blk['attn_w1'].T, 'ab1': blk['attn_b1'].reshape(1, MID * ATTN_M),
            'aw2': blk['attn_w2'].T, 'ab2': blk['attn_b2'].reshape(1, MID),
            'afw': blk['after_w'], 'afb': blk['after_b'].reshape(HIGH_C, 1),
        }
        q, kvx = _qkv(points, xhp, blk['before_w'],
                      blk['before_b'].reshape(MID, 1), blk['qkv_w'])
        table = kvx.reshape(B * N, 128)
        nsplit = 2 if blk_i == 0 else 4
        nblk = NROWB // nsplit
        parts = []
        for p in range(nsplit):
            gat = _sc_gather(table, part_idx(p, nsplit), 128)
            parts.append(_attn(points, q, gat, xhp, w, p * nblk, nblk))
        points = jnp.concatenate(parts, axis=2)

    return points


# speedup vs baseline: 1.0648x; 1.0648x over previous
import functools

import jax
import jax.numpy as jnp
from jax import lax
from jax.experimental import pallas as pl
from jax.experimental.pallas import tpu as pltpu
from jax.experimental.pallas import tpu_sc as plsc

B, S, N = 8, 512, 2048
LOW_C, HIGH_C = 256, 128
MID = HIGH_C // 4
POS_H = 64
ATTN_M = 4
K_NEI = 16
NBLK = 512
NROWB = N // NBLK

_INF = float('inf')


def _interp_body(plow_ref, upw_ref, upb_ref, xlr_ref, xh_ref, ph_ref, out_ref):
    plf = jnp.maximum(
        jnp.dot(upw_ref[...], plow_ref[...], preferred_element_type=jnp.float32)
        + upb_ref[...], 0.0)
    xlr = xlr_ref[...]
    xh = xh_ref[...]
    rn = jnp.sum(xlr * xlr, axis=1, keepdims=True)
    cn = jnp.sum(xh * xh, axis=0, keepdims=True)
    d2 = rn + cn - 2.0 * jnp.dot(xlr, xh, preferred_element_type=jnp.float32)
    row = lax.broadcasted_iota(jnp.int32, (S, N), 0)
    wt_u = jnp.zeros((S, N), jnp.float32)
    norm = jnp.zeros((1, N), jnp.float32)
    for _ in range(3):
        m = jnp.min(d2, axis=0, keepdims=True)
        idx = jnp.min(jnp.where(d2 == m, row, S), axis=0, keepdims=True)
        sel = row == idx
        d2 = jnp.where(sel, _INF, d2)
        rec = 1.0 / (m + 1e-8)
        norm = norm + rec
        wt_u = wt_u + jnp.where(sel, rec, 0.0)
    wt = wt_u / norm
    out_ref[...] = (
        jnp.dot(plf, wt, preferred_element_type=jnp.float32) + ph_ref[...])


def _interp(plow, upw, upb, xlr, xh, ph):
    return pl.pallas_call(
        _interp_body,
        grid=(B,),
        in_specs=[
            pl.BlockSpec((None, LOW_C, S), lambda b: (b, 0, 0)),
            pl.BlockSpec((HIGH_C, LOW_C), lambda b: (0, 0)),
            pl.BlockSpec((HIGH_C, 1), lambda b: (0, 0)),
            pl.BlockSpec((None, S, 3), lambda b: (b, 0, 0)),
            pl.BlockSpec((None, 3, N), lambda b: (b, 0, 0)),
            pl.BlockSpec((None, HIGH_C, N), lambda b: (b, 0, 0)),
        ],
        out_specs=pl.BlockSpec((None, HIGH_C, N), lambda b: (b, 0, 0)),
        out_shape=jax.ShapeDtypeStruct((B, HIGH_C, N), jnp.float32),
    )(plow, upw, upb, xlr, xh, ph)


KNB = 256


def _knn_body(xb_ref, xf_ref, out_ref):
    b = pl.program_id(0)
    xb = xb_ref[...]
    xf = xf_ref[...]
    rn = jnp.sum(xb * xb, axis=1, keepdims=True)
    cn = jnp.sum(xf * xf, axis=0, keepdims=True)
    d2 = rn + cn - 2.0 * jnp.dot(xb, xf, preferred_element_type=jnp.float32)
    lane = lax.broadcasted_iota(jnp.int32, (KNB, N), 1)
    bits = lax.bitcast_convert_type(d2, jnp.int32)
    packed = lax.bitcast_convert_type((bits & ~2047) | lane, jnp.float32)
    cols = []
    for _ in range(K_NEI):
        m = jnp.min(packed, axis=1, keepdims=True)
        packed = jnp.where(packed == m, _INF, packed)
        cols.append(lax.bitcast_convert_type(m, jnp.int32) & 2047)
    idx = jnp.concatenate(cols, axis=1)
    out_ref[...] = idx + b * N


def _knn(xhT, xf, h):
    hb = N // 2 // KNB
    return pl.pallas_call(
        _knn_body,
        grid=(B, hb),
        in_specs=[
            pl.BlockSpec((None, KNB, 3), lambda b, r: (b, h * hb + r, 0)),
            pl.BlockSpec((None, 3, N), lambda b, r: (b, 0, 0)),
        ],
        out_specs=pl.BlockSpec((None, KNB, K_NEI), lambda b, r: (b, r, 0)),
        out_shape=jax.ShapeDtypeStruct((B, N // 2, K_NEI), jnp.int32),
    )(xhT, xf)


def _qkv_body(pts_ref, xhp_ref, bw_ref, bb_ref, qkvw_ref, q_ref, kvx_ref):
    x = (jnp.dot(bw_ref[...], pts_ref[...], preferred_element_type=jnp.float32)
         + bb_ref[...])
    qkv = jnp.dot(qkvw_ref[...], x, preferred_element_type=jnp.float32)
    q_ref[...] = qkv[:MID, :]
    kvt = jnp.transpose(qkv[MID:, :])
    kvx_ref[...] = jnp.concatenate(
        [kvt, xhp_ref[...], jnp.zeros((N, 128 - 2 * MID - 16), jnp.float32)],
        axis=1)


def _qkv(pts, xhp, beforew, beforeb, qkvw):
    return pl.pallas_call(
        _qkv_body,
        grid=(B,),
        in_specs=[
            pl.BlockSpec((None, HIGH_C, N), lambda b: (b, 0, 0)),
            pl.BlockSpec((None, N, 16), lambda b: (b, 0, 0)),
            pl.BlockSpec((MID, HIGH_C), lambda b: (0, 0)),
            pl.BlockSpec((MID, 1), lambda b: (0, 0)),
            pl.BlockSpec((3 * MID, MID), lambda b: (0, 0)),
        ],
        out_specs=[
            pl.BlockSpec((None, MID, N), lambda b: (b, 0, 0)),
            pl.BlockSpec((None, N, 128), lambda b: (b, 0, 0)),
        ],
        out_shape=[
            jax.ShapeDtypeStruct((B, MID, N), jnp.float32),
            jax.ShapeDtypeStruct((B, N, 128), jnp.float32),
        ],
    )(pts, xhp, beforew, beforeb, qkvw)


def _sc_gather(table, idx, d):
    m_tot = idx.shape[0]
    info = plsc.get_sparse_core_info()
    nw = info.num_cores * info.num_subcores
    per_w = m_tot // nw
    sub = 128
    nf = 4
    ch = sub * nf
    n_ch = per_w // ch
    mesh = plsc.VectorSubcoreMesh(core_axis_name="c", subcore_axis_name="s")

    @functools.partial(
        pl.kernel, mesh=mesh,
        compiler_params=pltpu.CompilerParams(use_tc_tiling_on_sc=True),
        out_type=jax.ShapeDtypeStruct((m_tot, d), jnp.float32),
        scratch_types=[
            pltpu.VMEM((ch,), jnp.int32),
            pltpu.VMEM((ch, d), jnp.float32),
            pltpu.SemaphoreType.DMA,
        ],
    )
    def k(table_hbm, idx_hbm, out_hbm, idx_v, rows_v, sem):
        wid = lax.axis_index("s") * info.num_cores + lax.axis_index("c")
        base = wid * per_w

        def body(i, carry):
            off = base + i * ch
            pltpu.sync_copy(idx_hbm.at[pl.ds(off, ch)], idx_v)
            copies = [
                pltpu.async_copy(table_hbm.at[idx_v.at[pl.ds(f * sub, sub)]],
                                 rows_v.at[pl.ds(f * sub, sub)], sem)
                for f in range(nf)
            ]
            for c in copies:
                c.wait()
            pltpu.sync_copy(rows_v, out_hbm.at[pl.ds(off, ch)])
            return carry

        lax.fori_loop(0, n_ch, body, 0)

    return k(table, idx)


def _attn_body(pts_ref, q_ref, g_ref, xhp_ref,
               pw1_ref, pb1_ref, pw2_ref, pb2_ref,
               aw1_ref, ab1_ref, aw2_ref, ab2_ref,
               afw_ref, afb_ref, out_ref):
    rows = NBLK * K_NEI
    g = g_ref[...]
    kg = g[:, :MID]
    vg = g[:, MID:2 * MID]
    xyzg = g[:, 2 * MID:2 * MID + 16]
    xh_rep = jnp.broadcast_to(
        xhp_ref[...][:, None, :], (NBLK, K_NEI, 16)).reshape(rows, 16)
    qt = jnp.transpose(q_ref[...])
    q_rep = jnp.broadcast_to(
        qt[:, None, :], (NBLK, K_NEI, MID)).reshape(rows, MID)
    dxyz = xh_rep - xyzg
    h = jnp.maximum(
        jnp.dot(dxyz, pw1_ref[...], preferred_element_type=jnp.float32)
        + pb1_ref[...], 0.0)
    rel = (jnp.dot(h, pw2_ref[...], preferred_element_type=jnp.float32)
           + pb2_ref[...])
    sim_in = q_rep - kg + rel
    bf = jnp.bfloat16
    hid = jnp.maximum(
        jnp.dot(sim_in.astype(bf), aw1_ref[...].astype(bf),
                preferred_element_type=jnp.float32)
        + ab1_ref[...], 0.0)
    sim = (jnp.dot(hid.astype(bf), aw2_ref[...].astype(bf),
                   preferred_element_type=jnp.float32)
           + ab2_ref[...])
    e = jnp.exp(sim)
    vr = vg + rel
    num = jnp.sum((e * vr).reshape(NBLK, K_NEI, MID), axis=1)
    den = jnp.sum(e.reshape(NBLK, K_NEI, MID), axis=1)
    agg = jnp.transpose(num / den)
    out_ref[...] = (
        pts_ref[...]
        + jnp.dot(afw_ref[...], agg, preferred_element_type=jnp.float32)
        + afb_ref[...])


def _attn(pts, q, gat, xhp, w, start_blk, nblk):
    rows = NBLK * K_NEI

    def full(shp):
        return pl.BlockSpec(shp, lambda b, r: tuple(0 for _ in shp))

    return pl.pallas_call(
        _attn_body,
        grid=(B, nblk),
        in_specs=[
            pl.BlockSpec((None, HIGH_C, NBLK),
                         lambda b, r: (b, 0, start_blk + r)),
            pl.BlockSpec((None, MID, NBLK),
                         lambda b, r: (b, 0, start_blk + r)),
            pl.BlockSpec((rows, 128), lambda b, r: (b * nblk + r, 0)),
            pl.BlockSpec((None, NBLK, 16),
                         lambda b, r: (b, start_blk + r, 0)),
            full((16, POS_H)), full((1, POS_H)),
            full((POS_H, MID)), full((1, MID)),
            full((MID, MID * ATTN_M)), full((1, MID * ATTN_M)),
            full((MID * ATTN_M, MID)), full((1, MID)),
            full((HIGH_C, MID)), full((HIGH_C, 1)),
        ],
        out_specs=pl.BlockSpec((None, HIGH_C, NBLK), lambda b, r: (b, 0, r)),
        out_shape=jax.ShapeDtypeStruct((B, HIGH_C, nblk * NBLK), jnp.float32),
    )(pts, q, gat, xhp,
      w['pw1'], w['pb1'], w['pw2'], w['pb2'],
      w['aw1'], w['ab1'], w['aw2'], w['ab2'], w['afw'], w['afb'])


def kernel(xyz_low, xyz_high, points_low, points_high, params):
    f32 = jnp.float32
    xhT = jnp.transpose(xyz_high, (0, 2, 1))
    xlr = jnp.transpose(xyz_low, (0, 2, 1))

    inv = 1.0 / jnp.sqrt(f32(1.0 + 1e-5))
    scale = inv * params['up_bn_g']
    upw = params['up_w'] * scale[:, None]
    upb = (params['up_b'] * scale + params['up_bn_b']).reshape(HIGH_C, 1)

    points = _interp(points_low, upw, upb, xlr, xyz_high, points_high)

    xhp = jnp.pad(xhT, ((0, 0), (0, 0), (0, 13)))
    gidx_h = [_knn(xhT, xyz_high, h) for h in (0, 1)]

    def part_idx(p, nsplit):
        per = N // nsplit
        h, o = divmod(p * per, N // 2)
        return gidx_h[h][:, o:o + per, :].reshape(B * per * K_NEI)

    for blk_i, blk in enumerate(params['blocks']):
        w = {
            'pw1': jnp.pad(blk['pos_w1'], ((0, 0), (0, 13))).T,
            'pb1': blk['pos_b1'].reshape(1, POS_H),
            'pw2': blk['pos_w2'].T, 'pb2': blk['pos_b2'].reshape(1, MID),
            'aw1': blk['attn_w1'].T, 'ab1': blk['attn_b1'].reshape(1, MID * ATTN_M),
            'aw2': blk['attn_w2'].T, 'ab2': blk['attn_b2'].reshape(1, MID),
            'afw': blk['after_w'], 'afb': blk['after_b'].reshape(HIGH_C, 1),
        }
        q, kvx = _qkv(points, xhp, blk['before_w'],
                      blk['before_b'].reshape(MID, 1), blk['qkv_w'])
        table = kvx.reshape(B * N, 128)
        nsplit = 2 if blk_i == 0 else 4
        nblk = NROWB // nsplit
        parts = []
        for p in range(nsplit):
            gat = _sc_gather(table, part_idx(p, nsplit), 128)
            parts.append(_attn(points, q, gat, xhp, w, p * nblk, nblk))
        points = jnp.concatenate(parts, axis=2)

    return points
